# baseline (speedup 1.0000x reference)
import jax
import jax.numpy as jnp
from jax import lax
from jax.experimental import pallas as pl
from jax.experimental.pallas import tpu as pltpu
from jax.experimental.pallas import tpu_sc as plsc
from jax._src.pallas import mpmd as _plmpmd

QN = -127.0
QP = 127.0
MSB = 128

ROWS, COLS = 4096, 4096
BLOCK_ROWS = 512

NUM_WORKERS = 32
LANES = 16
CAP = 256
ROWS_PER_W = ROWS // NUM_WORKERS


def _dense_body(alpha_ref, w_ref, o_ref):
    a = alpha_ref[0]
    q = jnp.clip(w_ref[...] / a, QN, QP)
    o_ref[...] = jnp.round(q) * a


_dense_quantize = pl.pallas_call(
    _dense_body,
    grid=(ROWS // BLOCK_ROWS,),
    in_specs=[
        pl.BlockSpec(memory_space=pltpu.SMEM),
        pl.BlockSpec((BLOCK_ROWS, COLS), lambda i: (i, 0)),
    ],
    out_specs=pl.BlockSpec((BLOCK_ROWS, COLS), lambda i: (i, 0)),
    out_shape=jax.ShapeDtypeStruct((ROWS, COLS), jnp.float32),
)


def _flip_body(idx_hbm, wv_hbm, trow_hbm, tstart_hbm, tn_hbm, tcnt_hbm,
               alpha_hbm, wq_in, out_hbm,
               idx_v, wv_v, val_v, trow_v, tstart_v, tn_v, tcnt_v, alpha_v,
               rowbuf):
    del wq_in
    cid = lax.axis_index("c")
    sid = lax.axis_index("s")
    wid = sid * 2 + cid
    pltpu.sync_copy(idx_hbm.at[wid], idx_v.at[pl.ds(0, CAP)])
    pltpu.sync_copy(wv_hbm.at[wid], wv_v.at[pl.ds(0, CAP)])
    pltpu.sync_copy(trow_hbm.at[wid], trow_v.at[pl.ds(0, CAP)])
    pltpu.sync_copy(tstart_hbm.at[wid], tstart_v.at[pl.ds(0, CAP)])
    pltpu.sync_copy(tn_hbm.at[wid], tn_v.at[pl.ds(0, CAP)])
    pltpu.sync_copy(tcnt_hbm, tcnt_v.at[pl.ds(0, NUM_WORKERS)])
    pltpu.sync_copy(alpha_hbm, alpha_v)
    a = alpha_v[...]
    for j in range(CAP // LANES):
        w = wv_v[pl.ds(j * LANES, LANES)]
        sel = jnp.clip(w / a, QN, QP)
        flipped = (sel.astype(jnp.int32) ^ MSB).astype(jnp.float32)
        val_v[pl.ds(j * LANES, LANES)] = flipped * a
    tcnt = tcnt_v[pl.ds(wid, LANES)][0]

    def task(t, carry):
        r = trow_v[pl.ds(t, LANES)][0]
        s = tstart_v[pl.ds(t, LANES)][0]
        n = tn_v[pl.ds(t, LANES)][0]
        pltpu.sync_copy(out_hbm.at[r], rowbuf)

        def blend(jj, carry2):
            j = s + jj
            idx = idx_v[pl.ds(j, LANES)][0]
            v = val_v[pl.ds(j, LANES)][0]
            c = idx & (COLS - 1)
            c0 = (c // LANES) * LANES
            lane = c - c0
            vec = rowbuf[pl.ds(c0, LANES)]
            mask = lax.iota(jnp.int32, LANES) == lane
            rowbuf[pl.ds(c0, LANES)] = jnp.where(mask, v, vec)
            return carry2

        lax.fori_loop(0, n, blend, 0)
        pltpu.sync_copy(rowbuf, out_hbm.at[r])
        return carry

    lax.fori_loop(0, tcnt, task, 0)


_flip_scatter = _plmpmd._mpmd_map(
    [(plsc.VectorSubcoreMesh(core_axis_name="c", subcore_axis_name="s"),
      _flip_body)],
    out_types=jax.ShapeDtypeStruct((ROWS, COLS), jnp.float32),
    input_output_aliases={7: 0},
    scratch_types=[
        pltpu.VMEM((CAP + LANES,), jnp.int32),
        pltpu.VMEM((CAP + LANES,), jnp.float32),
        pltpu.VMEM((CAP + LANES,), jnp.float32),
        pltpu.VMEM((CAP + LANES,), jnp.int32),
        pltpu.VMEM((CAP + LANES,), jnp.int32),
        pltpu.VMEM((CAP + LANES,), jnp.int32),
        pltpu.VMEM((NUM_WORKERS + LANES,), jnp.int32),
        pltpu.VMEM((LANES,), jnp.float32),
        pltpu.VMEM((COLS,), jnp.float32),
    ],
)


def kernel(weight, alpha, flip_idx):
    alpha_eff = jnp.maximum(alpha[0], 1e-4)
    wq = _dense_quantize(alpha_eff.reshape(1), weight)

    nf = flip_idx.shape[0]
    fi = jnp.sort(flip_idx)
    rows = fi >> 12
    cols = fi & (COLS - 1)
    wv = weight[rows, cols]
    bounds = jnp.searchsorted(rows, jnp.arange(NUM_WORKERS + 1) * ROWS_PER_W
                              ).astype(jnp.int32)
    pos = bounds[:NUM_WORKERS, None] + jnp.arange(CAP)[None, :]
    pos = jnp.minimum(pos, nf - 1)
    idx_mat = fi[pos]
    wv_mat = wv[pos]
    heads = jnp.concatenate(
        [jnp.ones((1,), jnp.int32), (rows[1:] != rows[:-1]).astype(jnp.int32)])
    tid = jnp.cumsum(heads) - 1
    ntask = 2 * CAP * NUM_WORKERS // 8
    arange_f = jnp.arange(nf, dtype=jnp.int32)
    tstart_g = jax.ops.segment_min(arange_f, tid, num_segments=ntask)
    tn_g = jax.ops.segment_sum(jnp.ones((nf,), jnp.int32), tid,
                               num_segments=ntask)
    trow_g = jax.ops.segment_min(rows.astype(jnp.int32), tid,
                                 num_segments=ntask)
    tb = jnp.searchsorted(trow_g, jnp.arange(NUM_WORKERS + 1) * ROWS_PER_W
                          ).astype(jnp.int32)
    tcnts = tb[1:] - tb[:-1]
    tpos = jnp.minimum(tb[:NUM_WORKERS, None] + jnp.arange(CAP)[None, :],
                       ntask - 1)
    trow_mat = trow_g[tpos]
    tstart_mat = tstart_g[tpos] - bounds[:NUM_WORKERS, None]
    tn_mat = tn_g[tpos]
    alpha_vec = jnp.full((LANES,), alpha_eff, jnp.float32)
    out = _flip_scatter(idx_mat, wv_mat, trow_mat, tstart_mat, tn_mat,
                        tcnts, alpha_vec, wq)
    return out


# speedup vs baseline: 3.7025x; 3.7025x over previous
import jax
import jax.numpy as jnp
from jax import lax
from jax.experimental import pallas as pl
from jax.experimental.pallas import tpu as pltpu
from jax.experimental.pallas import tpu_sc as plsc
from jax._src.pallas import mpmd as _plmpmd

QN = -127.0
QP = 127.0
MSB = 128

ROWS, COLS = 4096, 4096
BLOCK_ROWS = 512

NUM_WORKERS = 32
LANES = 16
CAP = 256
NPAD = 2048
ROWS_PER_W = ROWS // NUM_WORKERS


def _dense_body(alpha_ref, w_ref, o_ref):
    a = alpha_ref[0]
    q = jnp.clip(w_ref[...] / a, QN, QP)
    o_ref[...] = jnp.round(q) * a


_dense_quantize = pl.pallas_call(
    _dense_body,
    grid=(ROWS // BLOCK_ROWS,),
    in_specs=[
        pl.BlockSpec(memory_space=pltpu.SMEM),
        pl.BlockSpec((BLOCK_ROWS, COLS), lambda i: (i, 0)),
    ],
    out_specs=pl.BlockSpec((BLOCK_ROWS, COLS), lambda i: (i, 0)),
    out_shape=jax.ShapeDtypeStruct((ROWS, COLS), jnp.float32),
)


def _flip_body(idx_hbm, meta_hbm, alpha_hbm, w_hbm, wq_in, out_hbm,
               idx_v, meta_v, alpha_v, rowbuf, wrowbuf, sem):
    del wq_in
    cid = lax.axis_index("c")
    sid = lax.axis_index("s")
    wid = sid * 2 + cid
    pltpu.sync_copy(meta_hbm, meta_v.at[pl.ds(0, 3 * NUM_WORKERS)])
    pltpu.sync_copy(alpha_hbm, alpha_v)
    base8 = pl.multiple_of(meta_v[pl.ds(wid, LANES)][0], 8)
    off = meta_v[pl.ds(NUM_WORKERS + wid, LANES)][0]
    cnt = meta_v[pl.ds(2 * NUM_WORKERS + wid, LANES)][0]
    pltpu.sync_copy(idx_hbm.at[pl.ds(base8, CAP + 8)],
                    idx_v.at[pl.ds(0, CAP + 8)])
    a = alpha_v[...]

    def flip(j, carry):
        p = off + j
        idx = idx_v[pl.ds(p, LANES)][0]
        prev = idx_v[pl.ds(jnp.maximum(p - 1, 0), LANES)][0]
        nxt = idx_v[pl.ds(p + 1, LANES)][0]
        r = idx >> 12
        c = idx & (COLS - 1)
        head = jnp.logical_or(j == 0, (prev >> 12) != r)
        last = jnp.logical_or(j == cnt - 1, (nxt >> 12) != r)

        @pl.when(head)
        def _():
            gat = pltpu.async_copy(out_hbm.at[r], rowbuf.at[pl.ds(0, COLS)],
                                   sem)
            gat2 = pltpu.async_copy(w_hbm.at[r], wrowbuf.at[pl.ds(0, COLS)],
                                    sem)
            gat.wait()
            gat2.wait()

        c0 = (c // LANES) * LANES
        lane = c - c0
        wvec = wrowbuf[pl.ds(c0, LANES)]
        sel = jnp.minimum(jnp.maximum(wvec / a, QN), QP)
        vvec = (sel.astype(jnp.int32) ^ MSB).astype(jnp.float32) * a
        vec = rowbuf[pl.ds(c0, LANES)]
        mask = lax.iota(jnp.int32, LANES) == lane
        rowbuf[pl.ds(c0, LANES)] = jnp.where(mask, vvec, vec)

        @pl.when(last)
        def _():
            pltpu.sync_copy(rowbuf.at[pl.ds(0, COLS)], out_hbm.at[r])

        return carry

    lax.fori_loop(0, cnt, flip, 0)


_flip_scatter = _plmpmd._mpmd_map(
    [(plsc.VectorSubcoreMesh(core_axis_name="c", subcore_axis_name="s"),
      _flip_body)],
    out_types=jax.ShapeDtypeStruct((ROWS, COLS), jnp.float32),
    input_output_aliases={4: 0},
    scratch_types=[
        pltpu.VMEM((CAP + 8 + LANES,), jnp.int32),
        pltpu.VMEM((3 * NUM_WORKERS + LANES,), jnp.int32),
        pltpu.VMEM((LANES,), jnp.float32),
        pltpu.VMEM((COLS + LANES,), jnp.float32),
        pltpu.VMEM((COLS + LANES,), jnp.float32),
        pltpu.SemaphoreType.DMA,
    ],
)


def kernel(weight, alpha, flip_idx):
    alpha_eff = jnp.maximum(alpha[0], 1e-4)
    wq = _dense_quantize(alpha_eff.reshape(1), weight)

    nf = flip_idx.shape[0]
    fi = jnp.sort(flip_idx)
    rows = fi >> 12
    band = jnp.arange(NUM_WORKERS, dtype=jnp.int32) * ROWS_PER_W
    bounds = jnp.sum(rows[None, :] < band[:, None], axis=1,
                     dtype=jnp.int32)
    endb = jnp.concatenate([bounds[1:], jnp.full((1,), nf, jnp.int32)])
    cnts = endb - bounds
    base8 = (bounds // 8) * 8
    off = bounds - base8
    meta = jnp.concatenate([base8, off, cnts])
    idx_pad = jnp.concatenate(
        [fi, jnp.broadcast_to(fi[-1:], (NPAD - nf,))])
    alpha_vec = jnp.full((LANES,), alpha_eff, jnp.float32)
    out = _flip_scatter(idx_pad, meta, alpha_vec, weight, wq)
    return out


# speedup vs baseline: 7.0508x; 1.9043x over previous
import jax
import jax.numpy as jnp
from jax import lax
from jax.experimental import pallas as pl
from jax.experimental.pallas import tpu as pltpu
from jax.experimental.pallas import tpu_sc as plsc
from jax._src.pallas import mpmd as _plmpmd

QN = -127.0
QP = 127.0
MSB = 128

ROWS, COLS = 4096, 4096
BLOCK_ROWS = 512

NUM_WORKERS = 32
LANES = 16
CAP = 256
NPAD = 2048
ROWS_PER_W = ROWS // NUM_WORKERS


def _dense_body(alpha_ref, w_ref, o_ref):
    a = alpha_ref[0]
    q = jnp.clip(w_ref[...] / a, QN, QP)
    o_ref[...] = jnp.round(q) * a


_dense_quantize = pl.pallas_call(
    _dense_body,
    grid=(ROWS // BLOCK_ROWS,),
    in_specs=[
        pl.BlockSpec(memory_space=pltpu.SMEM),
        pl.BlockSpec((BLOCK_ROWS, COLS), lambda i: (i, 0)),
    ],
    out_specs=pl.BlockSpec((BLOCK_ROWS, COLS), lambda i: (i, 0)),
    out_shape=jax.ShapeDtypeStruct((ROWS, COLS), jnp.float32),
)


ROUND_MAGIC = 12582912.0


def _flip_body(idx_hbm, meta_hbm, alpha_hbm, w_hbm, wq_in, out_hbm,
               idx_v, meta_v, alpha_v, wslots, oslots, sem):
    del wq_in
    cid = lax.axis_index("c")
    sid = lax.axis_index("s")
    wid = sid * 2 + cid
    pltpu.sync_copy(meta_hbm, meta_v.at[pl.ds(0, 3 * NUM_WORKERS)])
    pltpu.sync_copy(alpha_hbm, alpha_v)
    base8 = pl.multiple_of(meta_v[pl.ds(wid, LANES)][0], 8)
    off = meta_v[pl.ds(NUM_WORKERS + wid, LANES)][0]
    cnt = meta_v[pl.ds(2 * NUM_WORKERS + wid, LANES)][0]
    pltpu.sync_copy(idx_hbm.at[pl.ds(base8, CAP + 8)],
                    idx_v.at[pl.ds(0, CAP + 8)])
    a = alpha_v[...]

    def flip_at(j):
        return idx_v[pl.ds(off + j, LANES)][0]

    def coords(idx):
        r = idx >> 12
        c = idx & (COLS - 1)
        cg = pl.multiple_of((c // LANES) * LANES, LANES)
        return r, c, cg

    def fire(j, carry):
        r, _, cg = coords(flip_at(j))
        pltpu.async_copy(w_hbm.at[r, pl.ds(cg, LANES)], wslots.at[j], sem)
        return carry

    lax.fori_loop(0, cnt, fire, 0)

    def drain(j, carry):
        pltpu.make_async_copy(w_hbm.at[0, pl.ds(0, LANES)], wslots.at[0],
                              sem).wait()
        return carry

    lax.fori_loop(0, cnt, drain, 0)

    def proc(j, h):
        idx = flip_at(j)
        prev = idx_v[pl.ds(jnp.maximum(off + j - 1, 0), LANES)][0]
        nxt = idx_v[pl.ds(off + j + 1, LANES)][0]
        head = jnp.logical_or(j == 0, (prev >> 4) != (idx >> 4))
        last = jnp.logical_or(j == cnt - 1, (nxt >> 4) != (idx >> 4))
        h = jnp.where(head, j, h)
        r, c, cg = coords(idx)
        lane = c - cg
        wgrp = wslots[j]
        sel = jnp.minimum(jnp.maximum(wgrp / a, QN), QP)
        vvec = (sel.astype(jnp.int32) ^ MSB).astype(jnp.float32) * a
        dense = ((sel + ROUND_MAGIC) - ROUND_MAGIC) * a

        @pl.when(head)
        def _():
            oslots[h] = dense

        cur = oslots[h]
        mask = lax.iota(jnp.int32, LANES) == lane
        oslots[h] = jnp.where(mask, vvec, cur)

        @pl.when(last)
        def _():
            pltpu.async_copy(oslots.at[h], out_hbm.at[r, pl.ds(cg, LANES)],
                             sem)

        return h

    lax.fori_loop(0, cnt, proc, jnp.int32(0))

    def sdrain(j, carry):
        idx = flip_at(j)
        nxt = idx_v[pl.ds(off + j + 1, LANES)][0]
        last = jnp.logical_or(j == cnt - 1, (nxt >> 4) != (idx >> 4))

        @pl.when(last)
        def _():
            pltpu.make_async_copy(w_hbm.at[0, pl.ds(0, LANES)], oslots.at[0],
                                  sem).wait()

        return carry

    lax.fori_loop(0, cnt, sdrain, 0)


_flip_scatter = _plmpmd._mpmd_map(
    [(plsc.VectorSubcoreMesh(core_axis_name="c", subcore_axis_name="s"),
      _flip_body)],
    out_types=jax.ShapeDtypeStruct((ROWS, COLS), jnp.float32),
    input_output_aliases={4: 0},
    scratch_types=[
        pltpu.VMEM((CAP + 8 + LANES,), jnp.int32),
        pltpu.VMEM((3 * NUM_WORKERS + LANES,), jnp.int32),
        pltpu.VMEM((LANES,), jnp.float32),
        pltpu.VMEM((CAP, LANES), jnp.float32),
        pltpu.VMEM((CAP, LANES), jnp.float32),
        pltpu.SemaphoreType.DMA,
    ],
)


def kernel(weight, alpha, flip_idx):
    alpha_eff = jnp.maximum(alpha[0], 1e-4)
    wq = _dense_quantize(alpha_eff.reshape(1), weight)

    nf = flip_idx.shape[0]
    fi = jnp.sort(flip_idx)
    rows = fi >> 12
    band = jnp.arange(NUM_WORKERS, dtype=jnp.int32) * ROWS_PER_W
    bounds = jnp.sum(rows[None, :] < band[:, None], axis=1,
                     dtype=jnp.int32)
    endb = jnp.concatenate([bounds[1:], jnp.full((1,), nf, jnp.int32)])
    cnts = endb - bounds
    base8 = (bounds // 8) * 8
    off = bounds - base8
    meta = jnp.concatenate([base8, off, cnts])
    idx_pad = jnp.concatenate(
        [fi, jnp.broadcast_to(fi[-1:], (NPAD - nf,))])
    alpha_vec = jnp.full((LANES,), alpha_eff, jnp.float32)
    out = _flip_scatter(idx_pad, meta, alpha_vec, weight, wq)
    return out
